# baseline (speedup 1.0000x reference)
import functools

import jax
import jax.numpy as jnp
from jax import lax
from jax.experimental import pallas as pl
from jax.experimental.pallas import tpu as pltpu, tpu_sc as plsc

_B = 128
_STEPS = 3
_RBLK = 5000
_SBLK = 2 * _RBLK
_HD = 128
_NW = 32
_NPAD = 50176
_CHUNK = _NPAD // _NW
_ROWS = _B // 16


def _tc_pass(x00_ref, x01_ref, x10_ref, x11_ref, bi_ref, w0_ref, w1_ref,
             ex_ref, s_ref, pu_ref):
    i = pl.program_id(0)

    @pl.when(i == 0)
    def _init():
        s_ref[...] = jnp.zeros(s_ref.shape, jnp.float32)
        pu_ref[...] = jnp.zeros(pu_ref.shape, jnp.float32)

    dn = (((1,), (1,)), ((), ()))
    hi = lax.Precision.HIGHEST
    w0 = w0_ref[...]
    w1 = w1_ref[...]
    iota_b = lax.broadcasted_iota(jnp.int32, (1, _B), 1)
    ones_row = jnp.ones((1, _RBLK), jnp.float32)

    xa0, xa1 = x00_ref[...], x01_ref[...]
    xb0, xb1 = x10_ref[...], x11_ref[...]

    sx_a = (lax.dot_general(xa0, w0, dn, preferred_element_type=jnp.float32,
                            precision=hi)
            + lax.dot_general(xa1, w1, dn, preferred_element_type=jnp.float32,
                              precision=hi))
    sx_b = (lax.dot_general(xb0, w0, dn, preferred_element_type=jnp.float32,
                            precision=hi)
            + lax.dot_general(xb1, w1, dn, preferred_element_type=jnp.float32,
                              precision=hi))
    ex_a = jnp.exp(sx_a)
    ex_b = jnp.exp(sx_b)
    ex_ref[0:_RBLK, :] = ex_a
    ex_ref[_RBLK:_SBLK, :] = ex_b

    wm_a = jnp.where(bi_ref[0:_RBLK, :] == iota_b, ex_a, 0.0)
    wm_b = jnp.where(bi_ref[_RBLK:_SBLK, :] == iota_b, ex_b, 0.0)

    s_ref[...] += (
        lax.dot_general(ones_row, wm_a, (((1,), (0,)), ((), ())),
                        preferred_element_type=jnp.float32, precision=hi)
        + lax.dot_general(ones_row, wm_b, (((1,), (0,)), ((), ())),
                          preferred_element_type=jnp.float32, precision=hi))
    dc = (((0,), (0,)), ((), ()))
    pu_ref[:, 0:_HD] += (
        lax.dot_general(wm_a, xa0, dc, preferred_element_type=jnp.float32)
        + lax.dot_general(wm_b, xb0, dc, preferred_element_type=jnp.float32))
    pu_ref[:, _HD:2 * _HD] += (
        lax.dot_general(wm_a, xa1, dc, preferred_element_type=jnp.float32)
        + lax.dot_general(wm_b, xb1, dc, preferred_element_type=jnp.float32))


def _sc_pass(ex_hbm, bi_hbm, s_hbm, pu_hbm, attn_hbm, pooled_hbm,
             ex_v, bi_v, attn_v, s_v, fac_v, pu_v, dma_sem):
    wid = lax.axis_index("s") * 2 + lax.axis_index("c")
    base = wid * _CHUNK

    pltpu.sync_copy(ex_hbm.at[pl.ds(base, _CHUNK)], ex_v)
    pltpu.sync_copy(bi_hbm.at[pl.ds(base, _CHUNK)], bi_v)
    pltpu.sync_copy(s_hbm, s_v)

    for j in range(_B // 16):
        sv = s_v[pl.ds(j * 16, 16)]
        fac_v[pl.ds(j * 16, 16)] = jnp.where(sv > 0.0, 1.0 / sv, 0.0)

    def body(i, _):
        sl = pl.ds(i * 16, 16)
        fg = plsc.load_gather(fac_v, [bi_v[sl]])
        attn_v[sl] = ex_v[sl] * fg
        return ()

    lax.fori_loop(0, _CHUNK // 16, body, (), unroll=4)
    pltpu.sync_copy(attn_v, attn_hbm.at[pl.ds(base, _CHUNK)])

    @pl.when(wid < 16)
    def _pooled():
        row0 = wid * _ROWS
        pltpu.sync_copy(pu_hbm.at[pl.ds(row0, _ROWS)], pu_v)
        for r in range(_ROWS):
            idx = jnp.full((16,), row0 + r, jnp.int32)
            fr = plsc.load_gather(fac_v, [idx])
            for c in range(0, 256, 16):
                pu_v[r, pl.ds(c, 16)] = pu_v[r, pl.ds(c, 16)] * fr
        pltpu.sync_copy(pu_v, pooled_hbm.at[pl.ds(row0, _ROWS)])


@jax.jit
def kernel(x, batch_indices, W_ih, W_hh, b_ih, b_hh, W_att, b_att):
    n, d = x.shape
    nblk = n // _SBLK
    w_x = W_att[:, :d].astype(jnp.float32)
    bi32 = batch_indices.astype(jnp.int32)
    bi = bi32.reshape(n, 1)

    ex, s, pu = pl.pallas_call(
        _tc_pass,
        grid=(nblk,),
        in_specs=[
            pl.BlockSpec((_RBLK, _HD), lambda i: (2 * i, 0)),
            pl.BlockSpec((_RBLK, _HD), lambda i: (2 * i, 1)),
            pl.BlockSpec((_RBLK, _HD), lambda i: (2 * i + 1, 0)),
            pl.BlockSpec((_RBLK, _HD), lambda i: (2 * i + 1, 1)),
            pl.BlockSpec((_SBLK, 1), lambda i: (i, 0)),
            pl.BlockSpec((1, _HD), lambda i: (0, 0)),
            pl.BlockSpec((1, _HD), lambda i: (0, 1)),
        ],
        out_specs=[
            pl.BlockSpec((_SBLK, 1), lambda i: (i, 0)),
            pl.BlockSpec((1, _B), lambda i: (0, 0)),
            pl.BlockSpec((_B, d), lambda i: (0, 0)),
        ],
        out_shape=[
            jax.ShapeDtypeStruct((n, 1), jnp.float32),
            jax.ShapeDtypeStruct((1, _B), jnp.float32),
            jax.ShapeDtypeStruct((_B, d), jnp.float32),
        ],
    )(x, x, x, x, bi, w_x, w_x)

    ex_pad = jnp.pad(ex.reshape(n), (0, _NPAD - n))
    bi_pad = jnp.pad(bi32, (0, _NPAD - n))
    s_flat = s.reshape(_B)

    sc = functools.partial(
        pl.kernel,
        mesh=plsc.VectorSubcoreMesh(core_axis_name="c", subcore_axis_name="s"),
        compiler_params=pltpu.CompilerParams(needs_layout_passes=False),
        out_type=[
            jax.ShapeDtypeStruct((_NPAD,), jnp.float32),
            jax.ShapeDtypeStruct((_B, d), jnp.float32),
        ],
        scratch_types=[
            pltpu.VMEM((_CHUNK,), jnp.float32),
            pltpu.VMEM((_CHUNK,), jnp.int32),
            pltpu.VMEM((_CHUNK,), jnp.float32),
            pltpu.VMEM((_B,), jnp.float32),
            pltpu.VMEM((_B,), jnp.float32),
            pltpu.VMEM((_ROWS, 256), jnp.float32),
            pltpu.SemaphoreType.DMA,
        ],
    )(_sc_pass)
    attn_pad, pooled = sc(ex_pad, bi_pad, s_flat, pu)

    attn = attn_pad[:n]
    attn_steps = jnp.broadcast_to(attn.reshape(1, n), (_STEPS, n))
    return pooled, attn_steps


# speedup vs baseline: 1.0264x; 1.0264x over previous
import functools

import jax
import jax.numpy as jnp
from jax import lax
from jax.experimental import pallas as pl
from jax.experimental.pallas import tpu as pltpu, tpu_sc as plsc

_B = 128
_STEPS = 3
_RBLK = 5000
_SBLK = 2 * _RBLK
_HD = 128
_NW = 32
_NPAD = 50176
_CHUNK = _NPAD // _NW
_ROWS = _B // 16


def _tc_pass(xa_ref, xb_ref, bi_ref, w_ref, ex_ref, s_ref, pu_ref):
    i = pl.program_id(0)

    @pl.when(i == 0)
    def _init():
        s_ref[...] = jnp.zeros(s_ref.shape, jnp.float32)
        pu_ref[...] = jnp.zeros(pu_ref.shape, jnp.float32)

    dn = (((1,), (1,)), ((), ()))
    hi = lax.Precision.HIGHEST
    w = w_ref[...]
    iota_b = lax.broadcasted_iota(jnp.int32, (1, _B), 1)
    ones_row = jnp.ones((1, _RBLK), jnp.float32)

    xa = xa_ref[...]
    xb = xb_ref[...]

    sx_a = lax.dot_general(xa, w, dn, preferred_element_type=jnp.float32,
                           precision=hi)
    sx_b = lax.dot_general(xb, w, dn, preferred_element_type=jnp.float32,
                           precision=hi)
    ex_a = jnp.exp(sx_a)
    ex_b = jnp.exp(sx_b)
    ex_ref[0:_RBLK, :] = ex_a
    ex_ref[_RBLK:_SBLK, :] = ex_b

    wm_a = jnp.where(bi_ref[0:_RBLK, :] == iota_b, ex_a, 0.0)
    wm_b = jnp.where(bi_ref[_RBLK:_SBLK, :] == iota_b, ex_b, 0.0)

    s_ref[...] += (
        lax.dot_general(ones_row, wm_a, (((1,), (0,)), ((), ())),
                        preferred_element_type=jnp.float32, precision=hi)
        + lax.dot_general(ones_row, wm_b, (((1,), (0,)), ((), ())),
                          preferred_element_type=jnp.float32, precision=hi))
    dc = (((0,), (0,)), ((), ()))
    pu_ref[...] += (
        lax.dot_general(wm_a, xa, dc, preferred_element_type=jnp.float32)
        + lax.dot_general(wm_b, xb, dc, preferred_element_type=jnp.float32))


def _sc_pass(ex_hbm, bi_hbm, s_hbm, pu_hbm, attn_hbm, pooled_hbm,
             ex_v, bi_v, attn_v, s_v, fac_v, pu_v, dma_sem):
    wid = lax.axis_index("s") * 2 + lax.axis_index("c")
    base = wid * _CHUNK

    pltpu.sync_copy(ex_hbm.at[pl.ds(base, _CHUNK)], ex_v)
    pltpu.sync_copy(bi_hbm.at[pl.ds(base, _CHUNK)], bi_v)
    pltpu.sync_copy(s_hbm, s_v)

    for j in range(_B // 16):
        sv = s_v[pl.ds(j * 16, 16)]
        fac_v[pl.ds(j * 16, 16)] = jnp.where(sv > 0.0, 1.0 / sv, 0.0)

    def body(i, _):
        sl = pl.ds(i * 16, 16)
        fg = plsc.load_gather(fac_v, [bi_v[sl]])
        attn_v[sl] = ex_v[sl] * fg
        return ()

    lax.fori_loop(0, _CHUNK // 16, body, (), unroll=4)
    pltpu.sync_copy(attn_v, attn_hbm.at[pl.ds(base, _CHUNK)])

    @pl.when(wid < 16)
    def _pooled():
        row0 = wid * _ROWS
        pltpu.sync_copy(pu_hbm.at[pl.ds(row0, _ROWS)], pu_v)
        for r in range(_ROWS):
            idx = jnp.full((16,), row0 + r, jnp.int32)
            fr = plsc.load_gather(fac_v, [idx])
            for c in range(0, 256, 16):
                pu_v[r, pl.ds(c, 16)] = pu_v[r, pl.ds(c, 16)] * fr
        pltpu.sync_copy(pu_v, pooled_hbm.at[pl.ds(row0, _ROWS)])


@jax.jit
def kernel(x, batch_indices, W_ih, W_hh, b_ih, b_hh, W_att, b_att):
    n, d = x.shape
    nblk = n // _SBLK
    w_x = W_att[:, :d].astype(jnp.float32)
    bi32 = batch_indices.astype(jnp.int32)
    bi = bi32.reshape(n, 1)

    ex, s, pu = pl.pallas_call(
        _tc_pass,
        grid=(nblk,),
        in_specs=[
            pl.BlockSpec((_RBLK, d), lambda i: (2 * i, 0)),
            pl.BlockSpec((_RBLK, d), lambda i: (2 * i + 1, 0)),
            pl.BlockSpec((_SBLK, 1), lambda i: (i, 0)),
            pl.BlockSpec((1, d), lambda i: (0, 0)),
        ],
        out_specs=[
            pl.BlockSpec((_SBLK, 1), lambda i: (i, 0)),
            pl.BlockSpec((1, _B), lambda i: (0, 0)),
            pl.BlockSpec((_B, d), lambda i: (0, 0)),
        ],
        out_shape=[
            jax.ShapeDtypeStruct((n, 1), jnp.float32),
            jax.ShapeDtypeStruct((1, _B), jnp.float32),
            jax.ShapeDtypeStruct((_B, d), jnp.float32),
        ],
    )(x, x, bi, w_x)

    ex_pad = jnp.pad(ex.reshape(n), (0, _NPAD - n))
    bi_pad = jnp.pad(bi32, (0, _NPAD - n))
    s_flat = s.reshape(_B)

    sc = functools.partial(
        pl.kernel,
        mesh=plsc.VectorSubcoreMesh(core_axis_name="c", subcore_axis_name="s"),
        compiler_params=pltpu.CompilerParams(needs_layout_passes=False),
        out_type=[
            jax.ShapeDtypeStruct((_NPAD,), jnp.float32),
            jax.ShapeDtypeStruct((_B, d), jnp.float32),
        ],
        scratch_types=[
            pltpu.VMEM((_CHUNK,), jnp.float32),
            pltpu.VMEM((_CHUNK,), jnp.int32),
            pltpu.VMEM((_CHUNK,), jnp.float32),
            pltpu.VMEM((_B,), jnp.float32),
            pltpu.VMEM((_B,), jnp.float32),
            pltpu.VMEM((_ROWS, 256), jnp.float32),
            pltpu.SemaphoreType.DMA,
        ],
    )(_sc_pass)
    attn_pad, pooled = sc(ex_pad, bi_pad, s_flat, pu)

    attn = attn_pad[:n]
    attn_steps = jnp.broadcast_to(attn.reshape(1, n), (_STEPS, n))
    return pooled, attn_steps


# speedup vs baseline: 1.0668x; 1.0393x over previous
import functools

import jax
import jax.numpy as jnp
from jax import lax
from jax.experimental import pallas as pl
from jax.experimental.pallas import tpu as pltpu, tpu_sc as plsc

_B = 128
_STEPS = 3
_SBLK = 10000
_NW = 32
_NPAD = 50176
_CHUNK = _NPAD // _NW
_ROWS = _B // 16


def _tc_pass(x_ref, bi_ref, w_ref, ex_ref, s_ref, pu_ref):
    i = pl.program_id(0)

    @pl.when(i == 0)
    def _init():
        s_ref[...] = jnp.zeros(s_ref.shape, jnp.float32)
        pu_ref[...] = jnp.zeros(pu_ref.shape, jnp.float32)

    xb = x_ref[...]
    sx = lax.dot_general(xb, w_ref[...], (((1,), (1,)), ((), ())),
                         preferred_element_type=jnp.float32,
                         precision=lax.Precision.HIGHEST)
    ex = jnp.exp(sx)
    ex_ref[...] = ex

    iota_b = lax.broadcasted_iota(jnp.int32, (1, _B), 1)
    wm = jnp.where(bi_ref[...] == iota_b, ex, 0.0)

    ones_row = jnp.ones((1, _SBLK), jnp.float32)
    s_ref[...] += lax.dot_general(
        ones_row, wm, (((1,), (0,)), ((), ())),
        preferred_element_type=jnp.float32, precision=lax.Precision.HIGHEST)
    pu_ref[...] += lax.dot_general(
        wm, xb, (((0,), (0,)), ((), ())),
        preferred_element_type=jnp.float32)


def _sc_pass(ex_hbm, bi_hbm, s_hbm, pu_hbm, attn_hbm, pooled_hbm,
             ex_v, bi_v, attn_v, s_v, fac_v, pu_v, dma_sem):
    wid = lax.axis_index("s") * 2 + lax.axis_index("c")
    base = wid * _CHUNK

    pltpu.sync_copy(ex_hbm.at[pl.ds(base, _CHUNK)], ex_v)
    pltpu.sync_copy(bi_hbm.at[pl.ds(base, _CHUNK)], bi_v)
    pltpu.sync_copy(s_hbm, s_v)

    for j in range(_B // 16):
        sv = s_v[pl.ds(j * 16, 16)]
        fac_v[pl.ds(j * 16, 16)] = jnp.where(sv > 0.0, 1.0 / sv, 0.0)

    def body(i, _):
        sl = pl.ds(i * 16, 16)
        fg = plsc.load_gather(fac_v, [bi_v[sl]])
        attn_v[sl] = ex_v[sl] * fg
        return ()

    lax.fori_loop(0, _CHUNK // 16, body, (), unroll=4)
    pltpu.sync_copy(attn_v, attn_hbm.at[pl.ds(base, _CHUNK)])

    @pl.when(wid < 16)
    def _pooled():
        row0 = wid * _ROWS
        pltpu.sync_copy(pu_hbm.at[pl.ds(row0, _ROWS)], pu_v)
        for r in range(_ROWS):
            idx = jnp.full((16,), row0 + r, jnp.int32)
            fr = plsc.load_gather(fac_v, [idx])
            for c in range(0, 256, 16):
                pu_v[r, pl.ds(c, 16)] = pu_v[r, pl.ds(c, 16)] * fr
        pltpu.sync_copy(pu_v, pooled_hbm.at[pl.ds(row0, _ROWS)])


@jax.jit
def kernel(x, batch_indices, W_ih, W_hh, b_ih, b_hh, W_att, b_att):
    n, d = x.shape
    nblk = n // _SBLK
    w_x = W_att[:, :d].astype(jnp.float32)
    bi32 = batch_indices.astype(jnp.int32)
    bi = bi32.reshape(n, 1)

    ex, s, pu = pl.pallas_call(
        _tc_pass,
        grid=(nblk,),
        in_specs=[
            pl.BlockSpec((_SBLK, d), lambda i: (i, 0)),
            pl.BlockSpec((_SBLK, 1), lambda i: (i, 0)),
            pl.BlockSpec((1, d), lambda i: (0, 0)),
        ],
        out_specs=[
            pl.BlockSpec((_SBLK, 1), lambda i: (i, 0)),
            pl.BlockSpec((1, _B), lambda i: (0, 0)),
            pl.BlockSpec((_B, d), lambda i: (0, 0)),
        ],
        out_shape=[
            jax.ShapeDtypeStruct((n, 1), jnp.float32),
            jax.ShapeDtypeStruct((1, _B), jnp.float32),
            jax.ShapeDtypeStruct((_B, d), jnp.float32),
        ],
    )(x, bi, w_x)

    ex_pad = jnp.pad(ex.reshape(n), (0, _NPAD - n))
    bi_pad = jnp.pad(bi32, (0, _NPAD - n))
    s_flat = s.reshape(_B)

    sc = functools.partial(
        pl.kernel,
        mesh=plsc.VectorSubcoreMesh(core_axis_name="c", subcore_axis_name="s"),
        compiler_params=pltpu.CompilerParams(needs_layout_passes=False),
        out_type=[
            jax.ShapeDtypeStruct((_NPAD,), jnp.float32),
            jax.ShapeDtypeStruct((_B, d), jnp.float32),
        ],
        scratch_types=[
            pltpu.VMEM((_CHUNK,), jnp.float32),
            pltpu.VMEM((_CHUNK,), jnp.int32),
            pltpu.VMEM((_CHUNK,), jnp.float32),
            pltpu.VMEM((_B,), jnp.float32),
            pltpu.VMEM((_B,), jnp.float32),
            pltpu.VMEM((_ROWS, 256), jnp.float32),
            pltpu.SemaphoreType.DMA,
        ],
    )(_sc_pass)
    attn_pad, pooled = sc(ex_pad, bi_pad, s_flat, pu)

    attn = attn_pad[:n]
    attn_steps = jnp.broadcast_to(attn.reshape(1, n), (_STEPS, n))
    return pooled, attn_steps
